# baseline (speedup 1.0000x reference)
import functools

import jax
import jax.numpy as jnp
from jax import lax
from jax.experimental import pallas as pl
from jax.experimental.pallas import tpu as pltpu
from jax.experimental.pallas import tpu_sc as plsc

NUM_HEROES = 1000000
EMBED_DIM = 32
BATCH = 16384
SEQ = 10

_info = plsc.get_sparse_core_info()
NC, NS = _info.num_cores, _info.num_subcores
NW = NC * NS
TOTAL = BATCH * SEQ
ROWS_PER_W = TOTAL // NW
CHUNK = 128
NCH = ROWS_PER_W // CHUNK


def _body(x_hbm, table_hbm, out_hbm, idx_v, buf, gsem):
    wid = lax.axis_index("s") * NC + lax.axis_index("c")
    pltpu.sync_copy(x_hbm.at[wid], idx_v)
    base = wid * ROWS_PER_W

    def step(j, carry):
        pltpu.async_copy(table_hbm.at[idx_v.at[j]], buf, gsem).wait()
        pltpu.sync_copy(buf, out_hbm.at[pl.ds(base + j * CHUNK, CHUNK)])
        return carry

    lax.fori_loop(0, NCH, step, 0)


@jax.jit
def kernel(x, table):
    x_blocks = x.reshape(NW, NCH, CHUNK)
    run = pl.kernel(
        _body,
        out_type=jax.ShapeDtypeStruct((TOTAL, EMBED_DIM), jnp.float32),
        mesh=plsc.VectorSubcoreMesh(core_axis_name="c", subcore_axis_name="s"),
        scratch_types=[
            pltpu.VMEM((NCH, CHUNK), jnp.int32),
            pltpu.VMEM((CHUNK, EMBED_DIM), jnp.float32),
            pltpu.SemaphoreType.DMA,
        ],
        compiler_params=pltpu.CompilerParams(use_tc_tiling_on_sc=False),
    )
    out = run(x_blocks, table)
    return out.reshape(BATCH, SEQ, EMBED_DIM)


# speedup vs baseline: 1.0455x; 1.0455x over previous
import functools

import jax
import jax.numpy as jnp
from jax import lax
from jax.experimental import pallas as pl
from jax.experimental.pallas import tpu as pltpu
from jax.experimental.pallas import tpu_sc as plsc

NUM_HEROES = 1000000
EMBED_DIM = 32
BATCH = 16384
SEQ = 10

_info = plsc.get_sparse_core_info()
NC, NS = _info.num_cores, _info.num_subcores
NW = NC * NS
TOTAL = BATCH * SEQ
ROWS_PER_W = TOTAL // NW
CHUNK = 512
NCH = ROWS_PER_W // CHUNK
NBUF = 2


def _body(x_hbm, table_hbm, out_hbm, idx_v, buf0, buf1, g0, g1, w0, w1):
    wid = lax.axis_index("s") * NC + lax.axis_index("c")
    pltpu.sync_copy(x_hbm.at[wid], idx_v)
    base = wid * ROWS_PER_W
    bufs = (buf0, buf1)
    gsems = (g0, g1)
    wsems = (w0, w1)

    gd = {}
    wd = {}
    for j in range(NCH + 1):
        if j < NCH:
            b = j % NBUF
            if j >= NBUF:
                wd[j - NBUF].wait()
            gd[j] = pltpu.async_copy(table_hbm.at[idx_v.at[j]], bufs[b],
                                     gsems[b])
        if j >= 1:
            k = j - 1
            gd[k].wait()
            wd[k] = pltpu.async_copy(
                bufs[k % NBUF], out_hbm.at[pl.ds(base + k * CHUNK, CHUNK)],
                wsems[k % NBUF])
    wd[NCH - 1].wait()
    wd[NCH - 2].wait()


@jax.jit
def kernel(x, table):
    x_blocks = x.reshape(NW, NCH, CHUNK)
    run = pl.kernel(
        _body,
        out_type=jax.ShapeDtypeStruct((TOTAL, EMBED_DIM), jnp.float32),
        mesh=plsc.VectorSubcoreMesh(core_axis_name="c", subcore_axis_name="s"),
        scratch_types=[
            pltpu.VMEM((NCH, CHUNK), jnp.int32),
            pltpu.VMEM((CHUNK, EMBED_DIM), jnp.float32),
            pltpu.VMEM((CHUNK, EMBED_DIM), jnp.float32),
            pltpu.SemaphoreType.DMA,
            pltpu.SemaphoreType.DMA,
            pltpu.SemaphoreType.DMA,
            pltpu.SemaphoreType.DMA,
        ],
        compiler_params=pltpu.CompilerParams(use_tc_tiling_on_sc=False),
    )
    out = run(x_blocks, table)
    return out.reshape(BATCH, SEQ, EMBED_DIM)
